# baseline (speedup 1.0000x reference)
import functools

import jax
import jax.numpy as jnp
from jax import lax
from jax.experimental import pallas as pl
from jax.experimental.pallas import tpu as pltpu
import jax.experimental.pallas.tpu_sc as plsc

D = 128
BN = 512
T = 128
BS = 128
G = 128
CH = 6272
WS = 256
NC = 2
NS = 16
BIG = 2 ** 30

_SC_MESH = dict(core_axis_name="c", subcore_axis_name="s")


def _cdiv(a, b):
    return -(-a // b)


def _gelu(x):
    return 0.5 * x * (1.0 + lax.erf(x * 0.7071067811865476))


def _stage1_body(n, x_ref, wq_ref, wc_ref, wg_ref, bq_ref, bc_ref, bg_ref,
                 wcen_ref, cb_ref, q_ref, c_ref, g_ref, ci_ref):
    k = pl.program_id(0)
    x = x_ref[...]
    rows = k * BN + lax.broadcasted_iota(jnp.int32, (BN, 1), 0)
    msk = rows < n
    c = jnp.where(msk, jnp.dot(x, wc_ref[...], preferred_element_type=jnp.float32) + bc_ref[...], 0.0)
    q_ref[...] = jnp.where(msk, jnp.dot(x, wq_ref[...], preferred_element_type=jnp.float32) + bq_ref[...], 0.0)
    c_ref[...] = c
    g_ref[...] = jnp.where(msk, jnp.dot(x, wg_ref[...], preferred_element_type=jnp.float32) + bg_ref[...], 0.0)
    ci_ref[...] = jnp.dot(c, wcen_ref[...], preferred_element_type=jnp.float32) + cb_ref[...]


def _stage1(features, wq, wc, wg, bq, bc, bg, wcen, cb, npad):
    n = features.shape[0]
    wspec = pl.BlockSpec((D, D), lambda k: (0, 0))
    bspec = pl.BlockSpec((1, D), lambda k: (0, 0))
    rspec = pl.BlockSpec((BN, D), lambda k: (k, 0))
    out = jax.ShapeDtypeStruct((npad, D), jnp.float32)
    return pl.pallas_call(
        functools.partial(_stage1_body, n),
        grid=(npad // BN,),
        in_specs=[rspec, wspec, wspec, wspec, bspec, bspec, bspec,
                  wspec, bspec],
        out_specs=[rspec, rspec, rspec, rspec],
        out_shape=[out, out, out, out],
    )(features, wq, wc, wg, bq, bc, bg, wcen, cb)


def _convinit_body(x_ref, w_ref, b_ref, o_ref):
    o_ref[...] = jnp.dot(x_ref[...], w_ref[...], preferred_element_type=jnp.float32) + b_ref[...]


def _convinit(ctx, wcen, cb):
    npad = ctx.shape[0]
    return pl.pallas_call(
        _convinit_body,
        grid=(npad // BN,),
        in_specs=[pl.BlockSpec((BN, D), lambda k: (k, 0)),
                  pl.BlockSpec((D, D), lambda k: (0, 0)),
                  pl.BlockSpec((1, D), lambda k: (0, 0))],
        out_specs=pl.BlockSpec((BN, D), lambda k: (k, 0)),
        out_shape=jax.ShapeDtypeStruct((npad, D), jnp.float32),
    )(ctx, wcen, cb)


def _msgmm_body(kid_ref, x_ref, w_ref, o_ref):
    o_ref[...] = jnp.dot(x_ref[...], w_ref[0], preferred_element_type=jnp.float32)


def _msgmm(xg, cw, kid, nb):
    p_alloc = nb * T + WS
    grid_spec = pltpu.PrefetchScalarGridSpec(
        num_scalar_prefetch=1,
        grid=(nb,),
        in_specs=[pl.BlockSpec((T, D), lambda b, kid: (b, 0)),
                  pl.BlockSpec((1, D, D), lambda b, kid: (kid[b], 0, 0))],
        out_specs=pl.BlockSpec((T, D), lambda b, kid: (b, 0)),
    )
    return pl.pallas_call(
        _msgmm_body,
        grid_spec=grid_spec,
        out_shape=jax.ShapeDtypeStruct((p_alloc, D), jnp.float32),
    )(kid, xg, cw)


def _stage2_body(n, lvl, first, with_ci, *refs):
    if first:
        (conv_ref, g_ref, lg_ref, lb_ref), rest = refs[:4], refs[4:]
        allin_ref = None
    else:
        (conv_ref, g_ref, lg_ref, lb_ref, allin_ref), rest = refs[:5], refs[5:]
    if with_ci:
        (wcen_ref, cb_ref) = rest[:2]
        rest = rest[2:]
        ctx_ref, allout_ref, ci_ref = rest
    else:
        ctx_ref, allout_ref = rest
    k = pl.program_id(0)
    e = _gelu(conv_ref[...])
    m = jnp.mean(e, axis=-1, keepdims=True)
    v = jnp.mean((e - m) ** 2, axis=-1, keepdims=True)
    y = (e - m) / jnp.sqrt(v + 1e-5) * lg_ref[...] + lb_ref[...]
    rows = k * BN + lax.broadcasted_iota(jnp.int32, (BN, 1), 0)
    y = jnp.where(rows < n, y, 0.0)
    ctx_ref[...] = y
    gl = g_ref[:, lvl:lvl + 1]
    if first:
        allout_ref[...] = y * gl
    else:
        allout_ref[...] = allin_ref[...] + y * gl
    if with_ci:
        ci_ref[...] = jnp.dot(y, wcen_ref[...], preferred_element_type=jnp.float32) + cb_ref[...]


def _stage2(n, lvl, conv, gates, lg, lb, allin, wcen=None, cb=None):
    npad = conv.shape[0]
    rspec = pl.BlockSpec((BN, D), lambda k: (k, 0))
    bspec = pl.BlockSpec((1, D), lambda k: (0, 0))
    wspec = pl.BlockSpec((D, D), lambda k: (0, 0))
    out = jax.ShapeDtypeStruct((npad, D), jnp.float32)
    first = allin is None
    with_ci = wcen is not None
    ins = [conv, gates, lg, lb]
    in_specs = [rspec, rspec, bspec, bspec]
    if not first:
        ins += [allin]
        in_specs += [rspec]
    if with_ci:
        ins += [wcen, cb]
        in_specs += [wspec, bspec]
    n_out = 3 if with_ci else 2
    return pl.pallas_call(
        functools.partial(_stage2_body, n, lvl, first, with_ci),
        grid=(npad // BN,),
        in_specs=in_specs,
        out_specs=[rspec] * n_out,
        out_shape=[out] * n_out,
    )(*ins)


def _pool_body(ctx_ref, bid_ref, acc_ref):
    k = pl.program_id(0)

    @pl.when(k == 0)
    def _():
        acc_ref[...] = jnp.zeros_like(acc_ref)

    x = ctx_ref[...]
    bid = bid_ref[...]
    m0 = (bid == 0)
    m1 = (bid == 1)
    s0 = jnp.sum(jnp.where(m0, x, 0.0), axis=0, keepdims=True)
    s1 = jnp.sum(jnp.where(m1, x, 0.0), axis=0, keepdims=True)
    c0 = jnp.sum(m0.astype(jnp.float32))
    c1 = jnp.sum(m1.astype(jnp.float32))
    acc_ref[0:1, :] += s0
    acc_ref[1:2, :] += s1
    acc_ref[2:3, :] += jnp.full((1, D), c0)
    acc_ref[3:4, :] += jnp.full((1, D), c1)


def _pool(ctx, bid2d):
    npad = ctx.shape[0]
    return pl.pallas_call(
        _pool_body,
        grid=(npad // BN,),
        in_specs=[pl.BlockSpec((BN, D), lambda k: (k, 0)),
                  pl.BlockSpec((BN, 1), lambda k: (k, 0))],
        out_specs=pl.BlockSpec((8, D), lambda k: (0, 0)),
        out_shape=jax.ShapeDtypeStruct((8, D), jnp.float32),
    )(ctx, bid2d)


def _final_body(q_ref, all_ref, ps_ref, g_ref, bid_ref, wh_ref, bh_ref,
                wp_ref, bp_ref, o_ref):
    s = ps_ref[...]
    pooled = _gelu(s[0:2, :] / s[2:4, :])
    bid = bid_ref[...]
    sel = jnp.where(bid == 0, pooled[0:1, :], pooled[1:2, :])
    ctx_tot = all_ref[...] + sel * g_ref[:, 3:4]
    y = jnp.dot(ctx_tot, wh_ref[...], preferred_element_type=jnp.float32) + bh_ref[...]
    z = q_ref[...] * y
    o_ref[...] = jnp.dot(z, wp_ref[...], preferred_element_type=jnp.float32) + bp_ref[...]


def _final(n, q, ctx_all, pstats, gates, bid2d, wh, bh2, wp, bp2):
    npad = q.shape[0]
    rspec = pl.BlockSpec((BN, D), lambda k: (k, 0))
    wspec = pl.BlockSpec((D, D), lambda k: (0, 0))
    bspec = pl.BlockSpec((1, D), lambda k: (0, 0))
    return pl.pallas_call(
        _final_body,
        grid=(npad // BN,),
        in_specs=[rspec, rspec, pl.BlockSpec((8, D), lambda k: (0, 0)), rspec,
                  pl.BlockSpec((BN, 1), lambda k: (k, 0)),
                  wspec, bspec, wspec, bspec],
        out_specs=rspec,
        out_shape=jax.ShapeDtypeStruct((n, D), jnp.float32),
    )(q, ctx_all, pstats, gates, bid2d, wh, bh2, wp, bp2)


def _gather_block(n, lane, tab_ref, src_ref, e0t_v, vlot_v, vhit_v,
                  idxr_v, idxg_v, sem_i, b, j):
    e0 = pl.multiple_of(e0t_v[pl.ds(b, 16)][0], 8)
    return pltpu.async_copy(src_ref.at[pl.ds(e0, T)], idxr_v.at[j], sem_i)


def _gather_body(p, nbp, n, tab_ref, src_ref, e0t_ref, vlot_ref, vhit_ref,
                 out_ref, e0t_v, vlot_v, vhit_v, idxr_v, idxg_v, rows_v,
                 sem_i, sem_g):
    wid = lax.axis_index("s") * NC + lax.axis_index("c")
    nb = p // T
    npb = nb // (NC * NS)
    nst = npb // 4
    tail = npb % 4
    lane = lax.iota(jnp.int32, 16)
    pltpu.sync_copy(e0t_ref, e0t_v)
    pltpu.sync_copy(vlot_ref, vlot_v)
    pltpu.sync_copy(vhit_ref, vhit_v)
    base = wid * npb

    def build(b, j):
        vlo = vlot_v[pl.ds(b, 16)][0]
        vhi = vhit_v[pl.ds(b, 16)][0]
        for u in range(T // 16):
            tv = u * 16 + lane
            v = idxr_v[j, pl.ds(u * 16, 16)]
            idxg_v[j, pl.ds(u * 16, 16)] = (
                jnp.where((tv >= vlo) & (tv < vhi), v, n))

    def step(s, carry):
        b0 = base + s * 4
        descs = [_gather_block(n, lane, tab_ref, src_ref, e0t_v, vlot_v,
                               vhit_v, idxr_v, idxg_v, sem_i, b0 + j, j)
                 for j in range(4)]
        for dsc in descs:
            dsc.wait()
        for j in range(4):
            build(b0 + j, j)
        gds = [pltpu.async_copy(tab_ref.at[idxg_v.at[j]],
                                rows_v.at[pl.ds(j * T, T)], sem_g)
               for j in range(4)]
        for dsc in gds:
            dsc.wait()
        pltpu.sync_copy(rows_v, out_ref.at[pl.ds(b0 * T, 4 * T)])
        return carry

    lax.fori_loop(0, nst, step, 0)
    for j in range(tail):
        b = base + nst * 4 + j
        _gather_block(n, lane, tab_ref, src_ref, e0t_v, vlot_v, vhit_v,
                      idxr_v, idxg_v, sem_i, b, j).wait()
        build(b, j)
        pltpu.async_copy(tab_ref.at[idxg_v.at[j]],
                         rows_v.at[pl.ds(j * T, T)], sem_g).wait()
        pltpu.sync_copy(rows_v.at[pl.ds(j * T, T)],
                        out_ref.at[pl.ds(b * T, T)])


def _sc_gather(ctx_tab, src_raw, e0tab, vlotab, vhitab, p, n):
    nbp = e0tab.shape[0]
    kfn = pl.kernel(
        functools.partial(_gather_body, p, nbp, n),
        out_type=jax.ShapeDtypeStruct((p, D), jnp.float32),
        mesh=plsc.VectorSubcoreMesh(**_SC_MESH),
        scratch_types=[pltpu.VMEM((nbp,), jnp.int32),
                       pltpu.VMEM((nbp,), jnp.int32),
                       pltpu.VMEM((nbp,), jnp.int32),
                       pltpu.VMEM((4, T), jnp.int32),
                       pltpu.VMEM((4, T), jnp.int32),
                       pltpu.VMEM((4 * T, D), jnp.float32),
                       pltpu.SemaphoreType.DMA,
                       pltpu.SemaphoreType.DMA],
    )
    return kfn(ctx_tab, src_raw, e0tab, vlotab, vhitab)


def _scatter_body(npad, nj, conv_ref, msg_ref, dst_ref, st_ref, ln_ref,
                  e0t_ref, out_ref, spm, st_v, ln_v, e0t_v, idxr_v, idxa_v,
                  rows_v, sem_m):
    core = lax.axis_index("c")
    t = lax.axis_index("s")
    lane = lax.iota(jnp.int32, 16)
    trash = CH + t
    nchunks = _cdiv(npad, CH)
    pltpu.sync_copy(e0t_ref, e0t_v)

    for r in range(nchunks):
        @pl.when(core == r % NC)
        def _():
            row0 = r * CH
            rows_here = min(CH, npad - row0)
            per_tile = rows_here // NS
            if r >= NC:
                plsc.subcore_barrier()
            pltpu.sync_copy(conv_ref.at[pl.ds(row0 + t * per_tile, per_tile)],
                            spm.at[pl.ds(t * per_tile, per_tile)])
            pltpu.sync_copy(st_ref.at[r, t], st_v)
            pltpu.sync_copy(ln_ref.at[r, t], ln_v)
            plsc.subcore_barrier()

            def koff(j, carry):
                a = st_v[pl.ds(j, 16)][0]
                ln = ln_v[pl.ds(j, 16)][0]
                bend = a + ln
                o0 = (a // 8) * 8
                nwin = jnp.where(ln > 0, (bend - o0 + WS - 1) // WS, 0)

                def win(i, c2):
                    o = pl.multiple_of(o0 + i * WS, 8)
                    b = o // T
                    e0b = e0t_v[pl.ds(b, 16)][0]
                    raw0 = pl.multiple_of(e0b + (o - b * T), 8)
                    mdsc = pltpu.async_copy(msg_ref.at[pl.ds(o, WS)], rows_v,
                                            sem_m)
                    pltpu.sync_copy(dst_ref.at[pl.ds(raw0, WS)], idxr_v)
                    for u in range(WS // 16):
                        slot = o + u * 16 + lane
                        dv = idxr_v[pl.ds(u * 16, 16)]
                        m = (slot >= a) & (slot < bend)
                        idxa_v[u // 8, pl.ds((u % 8) * 16, 16)] = (
                            jnp.where(m, dv - row0, trash))
                    mdsc.wait()
                    for w in range(WS // T):
                        pltpu.sync_copy(rows_v.at[pl.ds(w * T, T)],
                                        spm.at[idxa_v.at[w]], add=True)
                    return c2

                lax.fori_loop(0, nwin, win, 0)
                return carry

            lax.fori_loop(0, nj, koff, 0)
            plsc.subcore_barrier()
            pltpu.sync_copy(spm.at[pl.ds(t * per_tile, per_tile)],
                            out_ref.at[pl.ds(row0 + t * per_tile, per_tile)])


def _sc_scatter(conv_init, msg, dst_raw, st_tab, ln_tab, e0tab, npad, nj):
    njp = st_tab.shape[2]
    nbp = e0tab.shape[0]
    kfn = pl.kernel(
        functools.partial(_scatter_body, npad, nj),
        out_type=jax.ShapeDtypeStruct((npad, D), jnp.float32),
        mesh=plsc.VectorSubcoreMesh(**_SC_MESH),
        scratch_types=[pltpu.VMEM_SHARED((CH + 16, D), jnp.float32),
                       pltpu.VMEM((njp,), jnp.int32),
                       pltpu.VMEM((njp,), jnp.int32),
                       pltpu.VMEM((nbp,), jnp.int32),
                       pltpu.VMEM((WS,), jnp.int32),
                       pltpu.VMEM((WS // T, T), jnp.int32),
                       pltpu.VMEM((WS, D), jnp.float32),
                       pltpu.SemaphoreType.DMA],
    )
    return kfn(conv_init, msg, dst_raw, st_tab, ln_tab, e0tab)


def _edge_layout(src, dst, cnt, n, nchunks):
    e = src.shape[0]
    k = cnt.shape[0]
    c0 = k // 2
    e_nc = e - n
    nb = _cdiv(e_nc, T) + 2 * k
    nb = _cdiv(nb, 32) * 32
    p = nb * T
    nbp = _cdiv(nb + 16, 8) * 8

    cntn = cnt.at[c0].set(0)
    starts = jnp.concatenate([jnp.zeros((1,), jnp.int32),
                              jnp.cumsum(cnt)[:-1].astype(jnp.int32)])
    sa = (starts // 8) * 8
    dk = starts - sa
    nbk = jnp.where(cntn > 0, (dk + cntn + T - 1) // T, 0)
    blk_cum = jnp.cumsum(nbk)
    blk_start = blk_cum - nbk
    total_blocks = blk_cum[-1]
    b = jnp.arange(nb)
    k_b = jnp.searchsorted(blk_cum, b, side='right').astype(jnp.int32)
    k_bc = jnp.minimum(k_b, k - 1)
    j_b = b - blk_start[k_bc]
    e0_b = jnp.clip(sa[k_bc] + j_b * T, 0, (e // 8) * 8)
    vlo_b = jnp.clip(dk[k_bc] - j_b * T, 0, T)
    vhi_b = jnp.clip(dk[k_bc] + cntn[k_bc] - j_b * T, 0, T)
    vhi_b = jnp.where(b < total_blocks, vhi_b, 0)
    kid_b = k_bc.astype(jnp.int32)
    e0tab = jnp.pad(e0_b.astype(jnp.int32), (0, nbp - nb))
    vlotab = jnp.pad(vlo_b.astype(jnp.int32), (0, nbp - nb))
    vhitab = jnp.pad(vhi_b.astype(jnp.int32), (0, nbp - nb))

    cum = jnp.cumsum(cnt)
    k_e = jnp.sum(jnp.arange(e, dtype=jnp.int32)[:, None] >= cum[None, :-1],
                  axis=1).astype(jnp.int32)
    comb = k_e * (nchunks + 1) + dst // CH
    bounds = jnp.searchsorted(comb, jnp.arange(k * (nchunks + 1) + 1)).astype(jnp.int32)
    qidx = jnp.arange(k)[:, None] * (nchunks + 1) + jnp.arange(nchunks)[None, :]
    st_raw = bounds[qidx]
    ln_kr = bounds[qidx + 1] - st_raw
    ln_kr = ln_kr.at[c0].set(0)
    st_kr = blk_start[:, None] * T + (st_raw - sa[:, None])

    nj = _cdiv(k, 16)
    njp = _cdiv(nj + 16, 8) * 8
    kk = jnp.arange(nj * 16)
    kkc = jnp.minimum(kk, k - 1)
    st_tab = jnp.where((kk < k)[:, None], st_kr[kkc], 0)
    ln_tab = jnp.where((kk < k)[:, None], ln_kr[kkc], 0)
    st_tab = jnp.pad(st_tab.reshape(nj, 16, nchunks).transpose(2, 1, 0),
                     ((0, 0), (0, 0), (0, njp - nj)))
    ln_tab = jnp.pad(ln_tab.reshape(nj, 16, nchunks).transpose(2, 1, 0),
                     ((0, 0), (0, 0), (0, njp - nj)))

    src_raw = jnp.pad(src, (0, 160))
    dst_raw = jnp.pad(dst, (0, T + WS + 160), constant_values=BIG)
    return (src_raw, dst_raw, kid_b, st_tab, ln_tab, e0tab, vlotab, vhitab,
            nb, p, nj, c0)


def kernel(features, Wf, bf, cw0, cb0, lg0, lb0, cw1, cb1, lg1, lb1,
           cw2, cb2, lg2, lb2, Wh, bh, Wp, bp, src0, dst0, cnt0,
           src1, dst1, cnt1, src2, dst2, cnt2, batch_id):
    n = features.shape[0]
    npad = _cdiv(n, BN) * BN
    nchunks = _cdiv(npad, CH)

    ncol = Wf.shape[1]
    wq = Wf[:, :D]
    wc = Wf[:, D:2 * D]
    wg = jnp.pad(Wf[:, 2 * D:], ((0, 0), (0, 3 * D - ncol)))
    bq = bf[:D].reshape(1, D)
    bc = bf[D:2 * D].reshape(1, D)
    bg = jnp.pad(bf[2 * D:], (0, 3 * D - ncol)).reshape(1, D)

    q, ctx, gates, conv_init = _stage1(
        features, wq, wc, wg, bq, bc, bg,
        cw0[cw0.shape[0] // 2], cb0.reshape(1, D), npad)

    bid2d = jnp.pad(batch_id.astype(jnp.int32), (0, npad - n),
                    constant_values=2).reshape(npad, 1)

    cws = [cw0, cw1, cw2]
    cbs = [cb0, cb1, cb2]
    lgs = [lg0, lg1, lg2]
    lbs = [lb0, lb1, lb2]
    srcs = [src0, src1, src2]
    dsts = [dst0, dst1, dst2]
    cnts = [cnt0, cnt1, cnt2]

    ctx_all = None
    for l in range(3):
        src = srcs[l].astype(jnp.int32)
        dst = dsts[l].astype(jnp.int32)
        cnt = cnts[l].astype(jnp.int32)
        (src_raw, dst_raw, kid_b, st_tab, ln_tab, e0tab, vlotab, vhitab,
         nb, p, nj, c0) = _edge_layout(src, dst, cnt, n, nchunks)

        xg = _sc_gather(ctx, src_raw, e0tab, vlotab, vhitab, p, n)
        msg = _msgmm(xg, cws[l], kid_b, nb)
        conv = _sc_scatter(conv_init, msg, dst_raw, st_tab, ln_tab, e0tab,
                           npad, nj)
        if l < 2:
            wnext = cws[l + 1][cws[l + 1].shape[0] // 2]
            ctx, ctx_all, conv_init = _stage2(
                n, l, conv, gates, lgs[l].reshape(1, D),
                lbs[l].reshape(1, D), ctx_all, wnext,
                cbs[l + 1].reshape(1, D))
        else:
            ctx, ctx_all = _stage2(n, l, conv, gates,
                                   lgs[l].reshape(1, D),
                                   lbs[l].reshape(1, D), ctx_all)

    pstats = _pool(ctx, bid2d)
    out = _final(n, q, ctx_all, pstats, gates, bid2d,
                 Wh, bh.reshape(1, D), Wp, bp.reshape(1, D))
    return out


# speedup vs baseline: 1.0410x; 1.0410x over previous
import functools

import jax
import jax.numpy as jnp
from jax import lax
from jax.experimental import pallas as pl
from jax.experimental.pallas import tpu as pltpu
import jax.experimental.pallas.tpu_sc as plsc

D = 128
BN = 512
T = 128
BS = 128
G = 128
CH = 8192
WS = 128
NC = 2
NS = 16
BIG = 2 ** 30

_SC_MESH = dict(core_axis_name="c", subcore_axis_name="s")


def _cdiv(a, b):
    return -(-a // b)


def _gelu(x):
    return 0.5 * x * (1.0 + lax.erf(x * 0.7071067811865476))


def _stage1_body(n, x_ref, wq_ref, wc_ref, wg_ref, bq_ref, bc_ref, bg_ref,
                 wcen_ref, cb_ref, q_ref, c_ref, g_ref, ci_ref):
    k = pl.program_id(0)
    x = x_ref[...]
    rows = k * BN + lax.broadcasted_iota(jnp.int32, (BN, 1), 0)
    msk = rows < n
    c = jnp.where(msk, jnp.dot(x, wc_ref[...], preferred_element_type=jnp.float32) + bc_ref[...], 0.0)
    q_ref[...] = jnp.where(msk, jnp.dot(x, wq_ref[...], preferred_element_type=jnp.float32) + bq_ref[...], 0.0)
    c_ref[...] = c
    g_ref[...] = jnp.where(msk, jnp.dot(x, wg_ref[...], preferred_element_type=jnp.float32) + bg_ref[...], 0.0)
    ci_ref[...] = jnp.dot(c, wcen_ref[...], preferred_element_type=jnp.float32) + cb_ref[...]


def _stage1(features, wq, wc, wg, bq, bc, bg, wcen, cb, npad):
    n = features.shape[0]
    wspec = pl.BlockSpec((D, D), lambda k: (0, 0))
    bspec = pl.BlockSpec((1, D), lambda k: (0, 0))
    rspec = pl.BlockSpec((BN, D), lambda k: (k, 0))
    out = jax.ShapeDtypeStruct((npad, D), jnp.float32)
    return pl.pallas_call(
        functools.partial(_stage1_body, n),
        grid=(npad // BN,),
        in_specs=[rspec, wspec, wspec, wspec, bspec, bspec, bspec,
                  wspec, bspec],
        out_specs=[rspec, rspec, rspec, rspec],
        out_shape=[out, out, out, out],
    )(features, wq, wc, wg, bq, bc, bg, wcen, cb)


def _convinit_body(x_ref, w_ref, b_ref, o_ref):
    o_ref[...] = jnp.dot(x_ref[...], w_ref[...], preferred_element_type=jnp.float32) + b_ref[...]


def _convinit(ctx, wcen, cb):
    npad = ctx.shape[0]
    return pl.pallas_call(
        _convinit_body,
        grid=(npad // BN,),
        in_specs=[pl.BlockSpec((BN, D), lambda k: (k, 0)),
                  pl.BlockSpec((D, D), lambda k: (0, 0)),
                  pl.BlockSpec((1, D), lambda k: (0, 0))],
        out_specs=pl.BlockSpec((BN, D), lambda k: (k, 0)),
        out_shape=jax.ShapeDtypeStruct((npad, D), jnp.float32),
    )(ctx, wcen, cb)


def _msgmm_body(kid_ref, x_ref, w_ref, o_ref):
    o_ref[...] = jnp.dot(x_ref[...], w_ref[0], preferred_element_type=jnp.float32)


def _msgmm(xg, cw, kid, nb):
    p_alloc = nb * T + WS
    grid_spec = pltpu.PrefetchScalarGridSpec(
        num_scalar_prefetch=1,
        grid=(nb,),
        in_specs=[pl.BlockSpec((T, D), lambda b, kid: (b, 0)),
                  pl.BlockSpec((1, D, D), lambda b, kid: (kid[b], 0, 0))],
        out_specs=pl.BlockSpec((T, D), lambda b, kid: (b, 0)),
    )
    return pl.pallas_call(
        _msgmm_body,
        grid_spec=grid_spec,
        out_shape=jax.ShapeDtypeStruct((p_alloc, D), jnp.float32),
    )(kid, xg, cw)


def _stage2_body(n, lvl, first, with_ci, *refs):
    if first:
        (conv_ref, g_ref, lg_ref, lb_ref), rest = refs[:4], refs[4:]
        allin_ref = None
    else:
        (conv_ref, g_ref, lg_ref, lb_ref, allin_ref), rest = refs[:5], refs[5:]
    if with_ci:
        (wcen_ref, cb_ref) = rest[:2]
        rest = rest[2:]
        ctx_ref, allout_ref, ci_ref = rest
    else:
        ctx_ref, allout_ref = rest
    k = pl.program_id(0)
    e = _gelu(conv_ref[...])
    m = jnp.mean(e, axis=-1, keepdims=True)
    v = jnp.mean((e - m) ** 2, axis=-1, keepdims=True)
    y = (e - m) / jnp.sqrt(v + 1e-5) * lg_ref[...] + lb_ref[...]
    rows = k * BN + lax.broadcasted_iota(jnp.int32, (BN, 1), 0)
    y = jnp.where(rows < n, y, 0.0)
    ctx_ref[...] = y
    gl = g_ref[:, lvl:lvl + 1]
    if first:
        allout_ref[...] = y * gl
    else:
        allout_ref[...] = allin_ref[...] + y * gl
    if with_ci:
        ci_ref[...] = jnp.dot(y, wcen_ref[...], preferred_element_type=jnp.float32) + cb_ref[...]


def _stage2(n, lvl, conv, gates, lg, lb, allin, wcen=None, cb=None):
    npad = conv.shape[0]
    rspec = pl.BlockSpec((BN, D), lambda k: (k, 0))
    bspec = pl.BlockSpec((1, D), lambda k: (0, 0))
    wspec = pl.BlockSpec((D, D), lambda k: (0, 0))
    out = jax.ShapeDtypeStruct((npad, D), jnp.float32)
    first = allin is None
    with_ci = wcen is not None
    ins = [conv, gates, lg, lb]
    in_specs = [rspec, rspec, bspec, bspec]
    if not first:
        ins += [allin]
        in_specs += [rspec]
    if with_ci:
        ins += [wcen, cb]
        in_specs += [wspec, bspec]
    n_out = 3 if with_ci else 2
    return pl.pallas_call(
        functools.partial(_stage2_body, n, lvl, first, with_ci),
        grid=(npad // BN,),
        in_specs=in_specs,
        out_specs=[rspec] * n_out,
        out_shape=[out] * n_out,
    )(*ins)


def _pool_body(ctx_ref, bid_ref, acc_ref):
    k = pl.program_id(0)

    @pl.when(k == 0)
    def _():
        acc_ref[...] = jnp.zeros_like(acc_ref)

    x = ctx_ref[...]
    bid = bid_ref[...]
    m0 = (bid == 0)
    m1 = (bid == 1)
    s0 = jnp.sum(jnp.where(m0, x, 0.0), axis=0, keepdims=True)
    s1 = jnp.sum(jnp.where(m1, x, 0.0), axis=0, keepdims=True)
    c0 = jnp.sum(m0.astype(jnp.float32))
    c1 = jnp.sum(m1.astype(jnp.float32))
    acc_ref[0:1, :] += s0
    acc_ref[1:2, :] += s1
    acc_ref[2:3, :] += jnp.full((1, D), c0)
    acc_ref[3:4, :] += jnp.full((1, D), c1)


def _pool(ctx, bid2d):
    npad = ctx.shape[0]
    return pl.pallas_call(
        _pool_body,
        grid=(npad // BN,),
        in_specs=[pl.BlockSpec((BN, D), lambda k: (k, 0)),
                  pl.BlockSpec((BN, 1), lambda k: (k, 0))],
        out_specs=pl.BlockSpec((8, D), lambda k: (0, 0)),
        out_shape=jax.ShapeDtypeStruct((8, D), jnp.float32),
    )(ctx, bid2d)


def _final_body(q_ref, all_ref, ps_ref, g_ref, bid_ref, wh_ref, bh_ref,
                wp_ref, bp_ref, o_ref):
    s = ps_ref[...]
    pooled = _gelu(s[0:2, :] / s[2:4, :])
    bid = bid_ref[...]
    sel = jnp.where(bid == 0, pooled[0:1, :], pooled[1:2, :])
    ctx_tot = all_ref[...] + sel * g_ref[:, 3:4]
    y = jnp.dot(ctx_tot, wh_ref[...], preferred_element_type=jnp.float32) + bh_ref[...]
    z = q_ref[...] * y
    o_ref[...] = jnp.dot(z, wp_ref[...], preferred_element_type=jnp.float32) + bp_ref[...]


def _final(n, q, ctx_all, pstats, gates, bid2d, wh, bh2, wp, bp2):
    npad = q.shape[0]
    rspec = pl.BlockSpec((BN, D), lambda k: (k, 0))
    wspec = pl.BlockSpec((D, D), lambda k: (0, 0))
    bspec = pl.BlockSpec((1, D), lambda k: (0, 0))
    return pl.pallas_call(
        _final_body,
        grid=(npad // BN,),
        in_specs=[rspec, rspec, pl.BlockSpec((8, D), lambda k: (0, 0)), rspec,
                  pl.BlockSpec((BN, 1), lambda k: (k, 0)),
                  wspec, bspec, wspec, bspec],
        out_specs=rspec,
        out_shape=jax.ShapeDtypeStruct((n, D), jnp.float32),
    )(q, ctx_all, pstats, gates, bid2d, wh, bh2, wp, bp2)


def _gather_block(n, lane, tab_ref, src_ref, e0t_v, vlot_v, vhit_v,
                  idxr_v, idxg_v, sem_i, b, j):
    e0 = pl.multiple_of(e0t_v[pl.ds(b, 16)][0], 8)
    return pltpu.async_copy(src_ref.at[pl.ds(e0, T)], idxr_v.at[j], sem_i)


def _gather_body(p, nbp, n, tab_ref, src_ref, e0t_ref, vlot_ref, vhit_ref,
                 out_ref, e0t_v, vlot_v, vhit_v, idxr_v, idxg_v, rows_v,
                 sem_i, sem_g):
    wid = lax.axis_index("s") * NC + lax.axis_index("c")
    nb = p // T
    npb = nb // (NC * NS)
    nst = npb // 4
    tail = npb % 4
    lane = lax.iota(jnp.int32, 16)
    pltpu.sync_copy(e0t_ref, e0t_v)
    pltpu.sync_copy(vlot_ref, vlot_v)
    pltpu.sync_copy(vhit_ref, vhit_v)
    base = wid * npb

    def build(b, j):
        vlo = vlot_v[pl.ds(b, 16)][0]
        vhi = vhit_v[pl.ds(b, 16)][0]
        for u in range(T // 16):
            tv = u * 16 + lane
            v = idxr_v[j, pl.ds(u * 16, 16)]
            idxg_v[j, pl.ds(u * 16, 16)] = (
                jnp.where((tv >= vlo) & (tv < vhi), v, n))

    def step(s, carry):
        b0 = base + s * 4
        descs = [_gather_block(n, lane, tab_ref, src_ref, e0t_v, vlot_v,
                               vhit_v, idxr_v, idxg_v, sem_i, b0 + j, j)
                 for j in range(4)]
        for dsc in descs:
            dsc.wait()
        for j in range(4):
            build(b0 + j, j)
        gds = [pltpu.async_copy(tab_ref.at[idxg_v.at[j]],
                                rows_v.at[pl.ds(j * T, T)], sem_g)
               for j in range(4)]
        for dsc in gds:
            dsc.wait()
        pltpu.sync_copy(rows_v, out_ref.at[pl.ds(b0 * T, 4 * T)])
        return carry

    lax.fori_loop(0, nst, step, 0)
    for j in range(tail):
        b = base + nst * 4 + j
        _gather_block(n, lane, tab_ref, src_ref, e0t_v, vlot_v, vhit_v,
                      idxr_v, idxg_v, sem_i, b, j).wait()
        build(b, j)
        pltpu.async_copy(tab_ref.at[idxg_v.at[j]],
                         rows_v.at[pl.ds(j * T, T)], sem_g).wait()
        pltpu.sync_copy(rows_v.at[pl.ds(j * T, T)],
                        out_ref.at[pl.ds(b * T, T)])


def _sc_gather(ctx_tab, src_raw, e0tab, vlotab, vhitab, p, n):
    nbp = e0tab.shape[0]
    kfn = pl.kernel(
        functools.partial(_gather_body, p, nbp, n),
        out_type=jax.ShapeDtypeStruct((p, D), jnp.float32),
        mesh=plsc.VectorSubcoreMesh(**_SC_MESH),
        scratch_types=[pltpu.VMEM((nbp,), jnp.int32),
                       pltpu.VMEM((nbp,), jnp.int32),
                       pltpu.VMEM((nbp,), jnp.int32),
                       pltpu.VMEM((4, T), jnp.int32),
                       pltpu.VMEM((4, T), jnp.int32),
                       pltpu.VMEM((4 * T, D), jnp.float32),
                       pltpu.SemaphoreType.DMA,
                       pltpu.SemaphoreType.DMA],
    )
    return kfn(ctx_tab, src_raw, e0tab, vlotab, vhitab)


def _scatter_body(npad, nj, conv_ref, msg_ref, dst_ref, st_ref, ln_ref,
                  e0t_ref, out_ref, spm, st_v, ln_v, e0t_v, idxr_v, idxa_v,
                  rows_v, sem_m):
    core = lax.axis_index("c")
    t = lax.axis_index("s")
    lane = lax.iota(jnp.int32, 16)
    trash = CH + t
    nchunks = _cdiv(npad, CH)
    pltpu.sync_copy(e0t_ref, e0t_v)

    for r in range(nchunks):
        @pl.when(core == r % NC)
        def _():
            row0 = r * CH
            rows_here = min(CH, npad - row0)
            per_tile = rows_here // NS
            if r >= NC:
                plsc.subcore_barrier()
            pltpu.sync_copy(conv_ref.at[pl.ds(row0 + t * per_tile, per_tile)],
                            spm.at[pl.ds(t * per_tile, per_tile)])
            pltpu.sync_copy(st_ref.at[r, t], st_v)
            pltpu.sync_copy(ln_ref.at[r, t], ln_v)
            plsc.subcore_barrier()

            def koff(j, carry):
                a = st_v[pl.ds(j, 16)][0]
                ln = ln_v[pl.ds(j, 16)][0]
                bend = a + ln
                o0 = (a // 8) * 8
                nwin = jnp.where(ln > 0, (bend - o0 + WS - 1) // WS, 0)

                def win(i, c2):
                    o = pl.multiple_of(o0 + i * WS, 8)
                    b = o // T
                    e0b = e0t_v[pl.ds(b, 16)][0]
                    raw0 = pl.multiple_of(e0b + (o - b * T), 8)
                    mdsc = pltpu.async_copy(msg_ref.at[pl.ds(o, WS)], rows_v,
                                            sem_m)
                    pltpu.sync_copy(dst_ref.at[pl.ds(raw0, WS)], idxr_v)
                    for u in range(WS // 16):
                        slot = o + u * 16 + lane
                        dv = idxr_v[pl.ds(u * 16, 16)]
                        m = (slot >= a) & (slot < bend)
                        idxa_v[u // 8, pl.ds((u % 8) * 16, 16)] = (
                            jnp.where(m, dv - row0, trash))
                    mdsc.wait()
                    for w in range(WS // T):
                        pltpu.sync_copy(rows_v.at[pl.ds(w * T, T)],
                                        spm.at[idxa_v.at[w]], add=True)
                    return c2

                lax.fori_loop(0, nwin, win, 0)
                return carry

            lax.fori_loop(0, nj, koff, 0)
            plsc.subcore_barrier()
            pltpu.sync_copy(spm.at[pl.ds(t * per_tile, per_tile)],
                            out_ref.at[pl.ds(row0 + t * per_tile, per_tile)])


def _sc_scatter(conv_init, msg, dst_raw, st_tab, ln_tab, e0tab, npad, nj):
    njp = st_tab.shape[2]
    nbp = e0tab.shape[0]
    kfn = pl.kernel(
        functools.partial(_scatter_body, npad, nj),
        out_type=jax.ShapeDtypeStruct((npad, D), jnp.float32),
        mesh=plsc.VectorSubcoreMesh(**_SC_MESH),
        scratch_types=[pltpu.VMEM_SHARED((CH + 16, D), jnp.float32),
                       pltpu.VMEM((njp,), jnp.int32),
                       pltpu.VMEM((njp,), jnp.int32),
                       pltpu.VMEM((nbp,), jnp.int32),
                       pltpu.VMEM((WS,), jnp.int32),
                       pltpu.VMEM((WS // T, T), jnp.int32),
                       pltpu.VMEM((WS, D), jnp.float32),
                       pltpu.SemaphoreType.DMA],
    )
    return kfn(conv_init, msg, dst_raw, st_tab, ln_tab, e0tab)


def _edge_layout(src, dst, cnt, n, nchunks):
    e = src.shape[0]
    k = cnt.shape[0]
    c0 = k // 2
    e_nc = e - n
    nb = _cdiv(e_nc, T) + 2 * k
    nb = _cdiv(nb, 32) * 32
    p = nb * T
    nbp = _cdiv(nb + 16, 8) * 8

    cntn = cnt.at[c0].set(0)
    starts = jnp.concatenate([jnp.zeros((1,), jnp.int32),
                              jnp.cumsum(cnt)[:-1].astype(jnp.int32)])
    sa = (starts // 8) * 8
    dk = starts - sa
    nbk = jnp.where(cntn > 0, (dk + cntn + T - 1) // T, 0)
    blk_cum = jnp.cumsum(nbk)
    blk_start = blk_cum - nbk
    total_blocks = blk_cum[-1]
    b = jnp.arange(nb)
    k_b = jnp.searchsorted(blk_cum, b, side='right').astype(jnp.int32)
    k_bc = jnp.minimum(k_b, k - 1)
    j_b = b - blk_start[k_bc]
    e0_b = jnp.clip(sa[k_bc] + j_b * T, 0, (e // 8) * 8)
    vlo_b = jnp.clip(dk[k_bc] - j_b * T, 0, T)
    vhi_b = jnp.clip(dk[k_bc] + cntn[k_bc] - j_b * T, 0, T)
    vhi_b = jnp.where(b < total_blocks, vhi_b, 0)
    kid_b = k_bc.astype(jnp.int32)
    e0tab = jnp.pad(e0_b.astype(jnp.int32), (0, nbp - nb))
    vlotab = jnp.pad(vlo_b.astype(jnp.int32), (0, nbp - nb))
    vhitab = jnp.pad(vhi_b.astype(jnp.int32), (0, nbp - nb))

    cum = jnp.cumsum(cnt)
    k_e = jnp.sum(jnp.arange(e, dtype=jnp.int32)[:, None] >= cum[None, :-1],
                  axis=1).astype(jnp.int32)
    comb = k_e * (nchunks + 1) + dst // CH
    bounds = jnp.searchsorted(comb, jnp.arange(k * (nchunks + 1) + 1)).astype(jnp.int32)
    qidx = jnp.arange(k)[:, None] * (nchunks + 1) + jnp.arange(nchunks)[None, :]
    st_raw = bounds[qidx]
    ln_kr = bounds[qidx + 1] - st_raw
    ln_kr = ln_kr.at[c0].set(0)
    st_kr = blk_start[:, None] * T + (st_raw - sa[:, None])

    nj = _cdiv(k, 16)
    njp = _cdiv(nj + 16, 8) * 8
    kk = jnp.arange(nj * 16)
    kkc = jnp.minimum(kk, k - 1)
    st_tab = jnp.where((kk < k)[:, None], st_kr[kkc], 0)
    ln_tab = jnp.where((kk < k)[:, None], ln_kr[kkc], 0)
    st_tab = jnp.pad(st_tab.reshape(nj, 16, nchunks).transpose(2, 1, 0),
                     ((0, 0), (0, 0), (0, njp - nj)))
    ln_tab = jnp.pad(ln_tab.reshape(nj, 16, nchunks).transpose(2, 1, 0),
                     ((0, 0), (0, 0), (0, njp - nj)))

    src_raw = jnp.pad(src, (0, 160))
    dst_raw = jnp.pad(dst, (0, T + WS + 160), constant_values=BIG)
    return (src_raw, dst_raw, kid_b, st_tab, ln_tab, e0tab, vlotab, vhitab,
            nb, p, nj, c0)


def kernel(features, Wf, bf, cw0, cb0, lg0, lb0, cw1, cb1, lg1, lb1,
           cw2, cb2, lg2, lb2, Wh, bh, Wp, bp, src0, dst0, cnt0,
           src1, dst1, cnt1, src2, dst2, cnt2, batch_id):
    n = features.shape[0]
    npad = _cdiv(n, BN) * BN
    nchunks = _cdiv(npad, CH)

    ncol = Wf.shape[1]
    wq = Wf[:, :D]
    wc = Wf[:, D:2 * D]
    wg = jnp.pad(Wf[:, 2 * D:], ((0, 0), (0, 3 * D - ncol)))
    bq = bf[:D].reshape(1, D)
    bc = bf[D:2 * D].reshape(1, D)
    bg = jnp.pad(bf[2 * D:], (0, 3 * D - ncol)).reshape(1, D)

    q, ctx, gates, conv_init = _stage1(
        features, wq, wc, wg, bq, bc, bg,
        cw0[cw0.shape[0] // 2], cb0.reshape(1, D), npad)

    bid2d = jnp.pad(batch_id.astype(jnp.int32), (0, npad - n),
                    constant_values=2).reshape(npad, 1)

    cws = [cw0, cw1, cw2]
    cbs = [cb0, cb1, cb2]
    lgs = [lg0, lg1, lg2]
    lbs = [lb0, lb1, lb2]
    srcs = [src0, src1, src2]
    dsts = [dst0, dst1, dst2]
    cnts = [cnt0, cnt1, cnt2]

    ctx_all = None
    for l in range(3):
        src = srcs[l].astype(jnp.int32)
        dst = dsts[l].astype(jnp.int32)
        cnt = cnts[l].astype(jnp.int32)
        (src_raw, dst_raw, kid_b, st_tab, ln_tab, e0tab, vlotab, vhitab,
         nb, p, nj, c0) = _edge_layout(src, dst, cnt, n, nchunks)

        xg = _sc_gather(ctx, src_raw, e0tab, vlotab, vhitab, p, n)
        msg = _msgmm(xg, cws[l], kid_b, nb)
        conv = _sc_scatter(conv_init, msg, dst_raw, st_tab, ln_tab, e0tab,
                           npad, nj)
        if l < 2:
            wnext = cws[l + 1][cws[l + 1].shape[0] // 2]
            ctx, ctx_all, conv_init = _stage2(
                n, l, conv, gates, lgs[l].reshape(1, D),
                lbs[l].reshape(1, D), ctx_all, wnext,
                cbs[l + 1].reshape(1, D))
        else:
            ctx, ctx_all = _stage2(n, l, conv, gates,
                                   lgs[l].reshape(1, D),
                                   lbs[l].reshape(1, D), ctx_all)

    pstats = _pool(ctx, bid2d)
    out = _final(n, q, ctx_all, pstats, gates, bid2d,
                 Wh, bh.reshape(1, D), Wp, bp.reshape(1, D))
    return out


# speedup vs baseline: 1.0414x; 1.0003x over previous
import functools

import jax
import jax.numpy as jnp
from jax import lax
from jax.experimental import pallas as pl
from jax.experimental.pallas import tpu as pltpu
import jax.experimental.pallas.tpu_sc as plsc

D = 128
BN = 512
T = 128
CH = 8192
WS = 128
NC = 2
NS = 16
BIG = 2 ** 30

_SC_MESH = dict(core_axis_name="c", subcore_axis_name="s")


def _cdiv(a, b):
    return -(-a // b)


def _gelu(x):
    return 0.5 * x * (1.0 + lax.erf(x * 0.7071067811865476))


def _stage1_body(n, x_ref, wq_ref, wc_ref, wg_ref, bq_ref, bc_ref, bg_ref,
                 wcen_ref, cb_ref, q_ref, c_ref, g_ref, ci_ref):
    k = pl.program_id(0)
    x = x_ref[...]
    rows = k * BN + lax.broadcasted_iota(jnp.int32, (BN, 1), 0)
    msk = rows < n
    c = jnp.where(msk, jnp.dot(x, wc_ref[...], preferred_element_type=jnp.float32) + bc_ref[...], 0.0)
    q_ref[...] = jnp.where(msk, jnp.dot(x, wq_ref[...], preferred_element_type=jnp.float32) + bq_ref[...], 0.0)
    c_ref[...] = c
    g_ref[...] = jnp.where(msk, jnp.dot(x, wg_ref[...], preferred_element_type=jnp.float32) + bg_ref[...], 0.0)
    ci_ref[...] = jnp.dot(c, wcen_ref[...], preferred_element_type=jnp.float32) + cb_ref[...]


def _stage1(features, wq, wc, wg, bq, bc, bg, wcen, cb, npad):
    n = features.shape[0]
    wspec = pl.BlockSpec((D, D), lambda k: (0, 0))
    bspec = pl.BlockSpec((1, D), lambda k: (0, 0))
    rspec = pl.BlockSpec((BN, D), lambda k: (k, 0))
    out = jax.ShapeDtypeStruct((npad, D), jnp.float32)
    return pl.pallas_call(
        functools.partial(_stage1_body, n),
        grid=(npad // BN,),
        in_specs=[rspec, wspec, wspec, wspec, bspec, bspec, bspec,
                  wspec, bspec],
        out_specs=[rspec, rspec, rspec, rspec],
        out_shape=[out, out, out, out],
    )(features, wq, wc, wg, bq, bc, bg, wcen, cb)


def _msgmm_body(kid_ref, x_ref, w_ref, o_ref):
    o_ref[...] = jnp.dot(x_ref[...], w_ref[0], preferred_element_type=jnp.float32)


def _msgmm(xg, cw, kid, nb):
    p_alloc = nb * T + WS
    grid_spec = pltpu.PrefetchScalarGridSpec(
        num_scalar_prefetch=1,
        grid=(nb,),
        in_specs=[pl.BlockSpec((T, D), lambda b, kid: (b, 0)),
                  pl.BlockSpec((1, D, D), lambda b, kid: (kid[b], 0, 0))],
        out_specs=pl.BlockSpec((T, D), lambda b, kid: (b, 0)),
    )
    return pl.pallas_call(
        _msgmm_body,
        grid_spec=grid_spec,
        out_shape=jax.ShapeDtypeStruct((p_alloc, D), jnp.float32),
    )(kid, xg, cw)


def _stage2_body(n, lvl, first, with_ci, *refs):
    if first:
        (conv_ref, g_ref, lg_ref, lb_ref), rest = refs[:4], refs[4:]
        allin_ref = None
    else:
        (conv_ref, g_ref, lg_ref, lb_ref, allin_ref), rest = refs[:5], refs[5:]
    if with_ci:
        (wcen_ref, cb_ref) = rest[:2]
        rest = rest[2:]
        ctx_ref, allout_ref, ci_ref = rest
    else:
        ctx_ref, allout_ref = rest
    k = pl.program_id(0)
    e = _gelu(conv_ref[...])
    m = jnp.mean(e, axis=-1, keepdims=True)
    v = jnp.mean((e - m) ** 2, axis=-1, keepdims=True)
    y = (e - m) / jnp.sqrt(v + 1e-5) * lg_ref[...] + lb_ref[...]
    rows = k * BN + lax.broadcasted_iota(jnp.int32, (BN, 1), 0)
    y = jnp.where(rows < n, y, 0.0)
    ctx_ref[...] = y
    gl = g_ref[:, lvl:lvl + 1]
    if first:
        allout_ref[...] = y * gl
    else:
        allout_ref[...] = allin_ref[...] + y * gl
    if with_ci:
        ci_ref[...] = jnp.dot(y, wcen_ref[...], preferred_element_type=jnp.float32) + cb_ref[...]


def _stage2(n, lvl, conv, gates, lg, lb, allin, wcen=None, cb=None):
    npad = conv.shape[0]
    rspec = pl.BlockSpec((BN, D), lambda k: (k, 0))
    bspec = pl.BlockSpec((1, D), lambda k: (0, 0))
    wspec = pl.BlockSpec((D, D), lambda k: (0, 0))
    out = jax.ShapeDtypeStruct((npad, D), jnp.float32)
    first = allin is None
    with_ci = wcen is not None
    ins = [conv, gates, lg, lb]
    in_specs = [rspec, rspec, bspec, bspec]
    if not first:
        ins += [allin]
        in_specs += [rspec]
    if with_ci:
        ins += [wcen, cb]
        in_specs += [wspec, bspec]
    n_out = 3 if with_ci else 2
    return pl.pallas_call(
        functools.partial(_stage2_body, n, lvl, first, with_ci),
        grid=(npad // BN,),
        in_specs=in_specs,
        out_specs=[rspec] * n_out,
        out_shape=[out] * n_out,
    )(*ins)


def _pool_body(ctx_ref, bid_ref, acc_ref):
    k = pl.program_id(0)

    @pl.when(k == 0)
    def _():
        acc_ref[...] = jnp.zeros_like(acc_ref)

    x = ctx_ref[...]
    bid = bid_ref[...]
    m0 = (bid == 0)
    m1 = (bid == 1)
    s0 = jnp.sum(jnp.where(m0, x, 0.0), axis=0, keepdims=True)
    s1 = jnp.sum(jnp.where(m1, x, 0.0), axis=0, keepdims=True)
    c0 = jnp.sum(m0.astype(jnp.float32))
    c1 = jnp.sum(m1.astype(jnp.float32))
    acc_ref[0:1, :] += s0
    acc_ref[1:2, :] += s1
    acc_ref[2:3, :] += jnp.full((1, D), c0)
    acc_ref[3:4, :] += jnp.full((1, D), c1)


def _pool(ctx, bid2d):
    npad = ctx.shape[0]
    return pl.pallas_call(
        _pool_body,
        grid=(npad // BN,),
        in_specs=[pl.BlockSpec((BN, D), lambda k: (k, 0)),
                  pl.BlockSpec((BN, 1), lambda k: (k, 0))],
        out_specs=pl.BlockSpec((8, D), lambda k: (0, 0)),
        out_shape=jax.ShapeDtypeStruct((8, D), jnp.float32),
    )(ctx, bid2d)


def _final_body(q_ref, all_ref, ps_ref, g_ref, bid_ref, wh_ref, bh_ref,
                wp_ref, bp_ref, o_ref):
    s = ps_ref[...]
    pooled = _gelu(s[0:2, :] / s[2:4, :])
    bid = bid_ref[...]
    sel = jnp.where(bid == 0, pooled[0:1, :], pooled[1:2, :])
    ctx_tot = all_ref[...] + sel * g_ref[:, 3:4]
    y = jnp.dot(ctx_tot, wh_ref[...], preferred_element_type=jnp.float32) + bh_ref[...]
    z = q_ref[...] * y
    o_ref[...] = jnp.dot(z, wp_ref[...], preferred_element_type=jnp.float32) + bp_ref[...]


def _final(n, q, ctx_all, pstats, gates, bid2d, wh, bh2, wp, bp2):
    npad = q.shape[0]
    rspec = pl.BlockSpec((BN, D), lambda k: (k, 0))
    wspec = pl.BlockSpec((D, D), lambda k: (0, 0))
    bspec = pl.BlockSpec((1, D), lambda k: (0, 0))
    return pl.pallas_call(
        _final_body,
        grid=(npad // BN,),
        in_specs=[rspec, rspec, pl.BlockSpec((8, D), lambda k: (0, 0)), rspec,
                  pl.BlockSpec((BN, 1), lambda k: (k, 0)),
                  wspec, bspec, wspec, bspec],
        out_specs=rspec,
        out_shape=jax.ShapeDtypeStruct((n, D), jnp.float32),
    )(q, ctx_all, pstats, gates, bid2d, wh, bh2, wp, bp2)


def _gather_block(n, lane, tab_ref, src_ref, e0t_v, vlot_v, vhit_v,
                  idxr_v, idxg_v, sem_i, b, j):
    e0 = pl.multiple_of(e0t_v[pl.ds(b, 16)][0], 8)
    return pltpu.async_copy(src_ref.at[pl.ds(e0, T)], idxr_v.at[j], sem_i)


def _gather_body(p, nbp, n, tab_ref, src_ref, e0t_ref, vlot_ref, vhit_ref,
                 out_ref, e0t_v, vlot_v, vhit_v, idxr_v, idxg_v, rows_v,
                 sem_i, sem_g):
    wid = lax.axis_index("s") * NC + lax.axis_index("c")
    nb = p // T
    npb = nb // (NC * NS)
    nst = npb // 4
    tail = npb % 4
    lane = lax.iota(jnp.int32, 16)
    pltpu.sync_copy(e0t_ref, e0t_v)
    pltpu.sync_copy(vlot_ref, vlot_v)
    pltpu.sync_copy(vhit_ref, vhit_v)
    base = wid * npb

    def build(b, j):
        vlo = vlot_v[pl.ds(b, 16)][0]
        vhi = vhit_v[pl.ds(b, 16)][0]
        for u in range(T // 16):
            tv = u * 16 + lane
            v = idxr_v[j, pl.ds(u * 16, 16)]
            idxg_v[j, pl.ds(u * 16, 16)] = (
                jnp.where((tv >= vlo) & (tv < vhi), v, n))

    def step(s, carry):
        b0 = base + s * 4
        descs = [_gather_block(n, lane, tab_ref, src_ref, e0t_v, vlot_v,
                               vhit_v, idxr_v, idxg_v, sem_i, b0 + j, j)
                 for j in range(4)]
        for dsc in descs:
            dsc.wait()
        for j in range(4):
            build(b0 + j, j)
        gds = [pltpu.async_copy(tab_ref.at[idxg_v.at[j]],
                                rows_v.at[pl.ds(j * T, T)], sem_g)
               for j in range(4)]
        for dsc in gds:
            dsc.wait()
        pltpu.sync_copy(rows_v, out_ref.at[pl.ds(b0 * T, 4 * T)])
        return carry

    lax.fori_loop(0, nst, step, 0)
    for j in range(tail):
        b = base + nst * 4 + j
        _gather_block(n, lane, tab_ref, src_ref, e0t_v, vlot_v, vhit_v,
                      idxr_v, idxg_v, sem_i, b, j).wait()
        build(b, j)
        pltpu.async_copy(tab_ref.at[idxg_v.at[j]],
                         rows_v.at[pl.ds(j * T, T)], sem_g).wait()
        pltpu.sync_copy(rows_v.at[pl.ds(j * T, T)],
                        out_ref.at[pl.ds(b * T, T)])


def _sc_gather(ctx_tab, src_raw, e0tab, vlotab, vhitab, p, n):
    nbp = e0tab.shape[0]
    kfn = pl.kernel(
        functools.partial(_gather_body, p, nbp, n),
        out_type=jax.ShapeDtypeStruct((p, D), jnp.float32),
        mesh=plsc.VectorSubcoreMesh(**_SC_MESH),
        scratch_types=[pltpu.VMEM((nbp,), jnp.int32),
                       pltpu.VMEM((nbp,), jnp.int32),
                       pltpu.VMEM((nbp,), jnp.int32),
                       pltpu.VMEM((4, T), jnp.int32),
                       pltpu.VMEM((4, T), jnp.int32),
                       pltpu.VMEM((4 * T, D), jnp.float32),
                       pltpu.SemaphoreType.DMA,
                       pltpu.SemaphoreType.DMA],
    )
    return kfn(ctx_tab, src_raw, e0tab, vlotab, vhitab)


def _scatter_body(npad, nj, conv_ref, msg_ref, dst_ref, st_ref, ln_ref,
                  e0t_ref, out_ref, spm, st_v, ln_v, e0t_v, idxr_v, idxa_v,
                  rows_v, sem_m):
    core = lax.axis_index("c")
    t = lax.axis_index("s")
    lane = lax.iota(jnp.int32, 16)
    trash = CH + t
    nchunks = _cdiv(npad, CH)
    pltpu.sync_copy(e0t_ref, e0t_v)

    for r in range(nchunks):
        @pl.when(core == r % NC)
        def _():
            row0 = r * CH
            rows_here = min(CH, npad - row0)
            per_tile = rows_here // NS
            if r >= NC:
                plsc.subcore_barrier()
            pltpu.sync_copy(conv_ref.at[pl.ds(row0 + t * per_tile, per_tile)],
                            spm.at[pl.ds(t * per_tile, per_tile)])
            pltpu.sync_copy(st_ref.at[r, t], st_v)
            pltpu.sync_copy(ln_ref.at[r, t], ln_v)
            plsc.subcore_barrier()

            def koff(j, carry):
                a = st_v[pl.ds(j, 16)][0]
                ln = ln_v[pl.ds(j, 16)][0]
                bend = a + ln
                o0 = (a // 8) * 8
                nwin = jnp.where(ln > 0, (bend - o0 + WS - 1) // WS, 0)

                def win(i, c2):
                    o = pl.multiple_of(o0 + i * WS, 8)
                    b = o // T
                    e0b = e0t_v[pl.ds(b, 16)][0]
                    raw0 = pl.multiple_of(e0b + (o - b * T), 8)
                    mdsc = pltpu.async_copy(msg_ref.at[pl.ds(o, WS)], rows_v,
                                            sem_m)
                    pltpu.sync_copy(dst_ref.at[pl.ds(raw0, WS)], idxr_v)
                    for u in range(WS // 16):
                        slot = o + u * 16 + lane
                        dv = idxr_v[pl.ds(u * 16, 16)]
                        m = (slot >= a) & (slot < bend)
                        idxa_v[u // 8, pl.ds((u % 8) * 16, 16)] = (
                            jnp.where(m, dv - row0, trash))
                    mdsc.wait()
                    for w in range(WS // T):
                        pltpu.sync_copy(rows_v.at[pl.ds(w * T, T)],
                                        spm.at[idxa_v.at[w]], add=True)
                    return c2

                lax.fori_loop(0, nwin, win, 0)
                return carry

            lax.fori_loop(0, nj, koff, 0)
            plsc.subcore_barrier()
            pltpu.sync_copy(spm.at[pl.ds(t * per_tile, per_tile)],
                            out_ref.at[pl.ds(row0 + t * per_tile, per_tile)])


def _sc_scatter(conv_init, msg, dst_raw, st_tab, ln_tab, e0tab, npad, nj):
    njp = st_tab.shape[2]
    nbp = e0tab.shape[0]
    kfn = pl.kernel(
        functools.partial(_scatter_body, npad, nj),
        out_type=jax.ShapeDtypeStruct((npad, D), jnp.float32),
        mesh=plsc.VectorSubcoreMesh(**_SC_MESH),
        scratch_types=[pltpu.VMEM_SHARED((CH + 16, D), jnp.float32),
                       pltpu.VMEM((njp,), jnp.int32),
                       pltpu.VMEM((njp,), jnp.int32),
                       pltpu.VMEM((nbp,), jnp.int32),
                       pltpu.VMEM((WS,), jnp.int32),
                       pltpu.VMEM((WS // T, T), jnp.int32),
                       pltpu.VMEM((WS, D), jnp.float32),
                       pltpu.SemaphoreType.DMA],
    )
    return kfn(conv_init, msg, dst_raw, st_tab, ln_tab, e0tab)


def _edge_layout(src, dst, cnt, n, nchunks):
    e = src.shape[0]
    k = cnt.shape[0]
    c0 = k // 2
    e_nc = e - n
    nb = _cdiv(e_nc, T) + 2 * k
    nb = _cdiv(nb, 32) * 32
    p = nb * T
    nbp = _cdiv(nb + 16, 8) * 8

    cntn = cnt.at[c0].set(0)
    starts = jnp.concatenate([jnp.zeros((1,), jnp.int32),
                              jnp.cumsum(cnt)[:-1].astype(jnp.int32)])
    sa = (starts // 8) * 8
    dk = starts - sa
    nbk = jnp.where(cntn > 0, (dk + cntn + T - 1) // T, 0)
    blk_cum = jnp.cumsum(nbk)
    blk_start = blk_cum - nbk
    total_blocks = blk_cum[-1]
    b = jnp.arange(nb)
    k_b = jnp.searchsorted(blk_cum, b, side='right').astype(jnp.int32)
    k_bc = jnp.minimum(k_b, k - 1)
    j_b = b - blk_start[k_bc]
    e0_b = jnp.clip(sa[k_bc] + j_b * T, 0, (e // 8) * 8)
    vlo_b = jnp.clip(dk[k_bc] - j_b * T, 0, T)
    vhi_b = jnp.clip(dk[k_bc] + cntn[k_bc] - j_b * T, 0, T)
    vhi_b = jnp.where(b < total_blocks, vhi_b, 0)
    kid_b = k_bc.astype(jnp.int32)
    e0tab = jnp.pad(e0_b.astype(jnp.int32), (0, nbp - nb))
    vlotab = jnp.pad(vlo_b.astype(jnp.int32), (0, nbp - nb))
    vhitab = jnp.pad(vhi_b.astype(jnp.int32), (0, nbp - nb))

    cum = jnp.cumsum(cnt)
    k_e = jnp.sum(jnp.arange(e, dtype=jnp.int32)[:, None] >= cum[None, :-1],
                  axis=1).astype(jnp.int32)
    comb = k_e * (nchunks + 1) + dst // CH
    bounds = jnp.searchsorted(comb, jnp.arange(k * (nchunks + 1) + 1)).astype(jnp.int32)
    qidx = jnp.arange(k)[:, None] * (nchunks + 1) + jnp.arange(nchunks)[None, :]
    st_raw = bounds[qidx]
    ln_kr = bounds[qidx + 1] - st_raw
    ln_kr = ln_kr.at[c0].set(0)
    st_kr = blk_start[:, None] * T + (st_raw - sa[:, None])

    nj = _cdiv(k, 16)
    njp = _cdiv(nj + 16, 8) * 8
    kk = jnp.arange(nj * 16)
    kkc = jnp.minimum(kk, k - 1)
    st_tab = jnp.where((kk < k)[:, None], st_kr[kkc], 0)
    ln_tab = jnp.where((kk < k)[:, None], ln_kr[kkc], 0)
    st_tab = jnp.pad(st_tab.reshape(nj, 16, nchunks).transpose(2, 1, 0),
                     ((0, 0), (0, 0), (0, njp - nj)))
    ln_tab = jnp.pad(ln_tab.reshape(nj, 16, nchunks).transpose(2, 1, 0),
                     ((0, 0), (0, 0), (0, njp - nj)))

    src_raw = jnp.pad(src, (0, 160))
    dst_raw = jnp.pad(dst, (0, T + WS + 160), constant_values=BIG)
    return (src_raw, dst_raw, kid_b, st_tab, ln_tab, e0tab, vlotab, vhitab,
            nb, p, nj, c0)


def kernel(features, Wf, bf, cw0, cb0, lg0, lb0, cw1, cb1, lg1, lb1,
           cw2, cb2, lg2, lb2, Wh, bh, Wp, bp, src0, dst0, cnt0,
           src1, dst1, cnt1, src2, dst2, cnt2, batch_id):
    n = features.shape[0]
    npad = _cdiv(n, BN) * BN
    nchunks = _cdiv(npad, CH)

    ncol = Wf.shape[1]
    wq = Wf[:, :D]
    wc = Wf[:, D:2 * D]
    wg = jnp.pad(Wf[:, 2 * D:], ((0, 0), (0, 3 * D - ncol)))
    bq = bf[:D].reshape(1, D)
    bc = bf[D:2 * D].reshape(1, D)
    bg = jnp.pad(bf[2 * D:], (0, 3 * D - ncol)).reshape(1, D)

    q, ctx, gates, conv_init = _stage1(
        features, wq, wc, wg, bq, bc, bg,
        cw0[cw0.shape[0] // 2], cb0.reshape(1, D), npad)

    bid2d = jnp.pad(batch_id.astype(jnp.int32), (0, npad - n),
                    constant_values=2).reshape(npad, 1)

    cws = [cw0, cw1, cw2]
    cbs = [cb0, cb1, cb2]
    lgs = [lg0, lg1, lg2]
    lbs = [lb0, lb1, lb2]
    srcs = [src0, src1, src2]
    dsts = [dst0, dst1, dst2]
    cnts = [cnt0, cnt1, cnt2]

    ctx_all = None
    for l in range(3):
        src = srcs[l].astype(jnp.int32)
        dst = dsts[l].astype(jnp.int32)
        cnt = cnts[l].astype(jnp.int32)
        (src_raw, dst_raw, kid_b, st_tab, ln_tab, e0tab, vlotab, vhitab,
         nb, p, nj, c0) = _edge_layout(src, dst, cnt, n, nchunks)

        xg = _sc_gather(ctx, src_raw, e0tab, vlotab, vhitab, p, n)
        msg = _msgmm(xg, cws[l], kid_b, nb)
        conv = _sc_scatter(conv_init, msg, dst_raw, st_tab, ln_tab, e0tab,
                           npad, nj)
        if l < 2:
            wnext = cws[l + 1][cws[l + 1].shape[0] // 2]
            ctx, ctx_all, conv_init = _stage2(
                n, l, conv, gates, lgs[l].reshape(1, D),
                lbs[l].reshape(1, D), ctx_all, wnext,
                cbs[l + 1].reshape(1, D))
        else:
            ctx, ctx_all = _stage2(n, l, conv, gates,
                                   lgs[l].reshape(1, D),
                                   lbs[l].reshape(1, D), ctx_all)

    pstats = _pool(ctx, bid2d)
    out = _final(n, q, ctx_all, pstats, gates, bid2d,
                 Wh, bh.reshape(1, D), Wp, bp.reshape(1, D))
    return out
